# Initial kernel scaffold; baseline (speedup 1.0000x reference)
import jax
import jax.numpy as jnp
from jax.experimental import pallas as pl


def kernel(boxes, feat2, feat3, feat4, feat5):
    raise NotImplementedError("write your pallas kernel here")


# speedup vs baseline: 31.6613x; 31.6613x over previous
import functools

import jax
import jax.numpy as jnp
import numpy as np
from jax import lax
from jax.experimental import pallas as pl
from jax.experimental.pallas import tpu as pltpu
from jax.experimental.pallas import tpu_sc as plsc

_N = 1000
_NPAD = 1024
_NW = 32
_C = 256
_S = 49
_SPAD = 56
_GROUP_BASES = (0, 16, 32, 40)


def _body(bx, lvl, gy, gx, f2, f3, f4, f5, out,
          bx_v, lvl_v, gy_v, gx_v, idx_b, wt_b, crn, out_v, sem):
    wid = lax.axis_index("c") * 16 + lax.axis_index("s")

    pltpu.sync_copy(bx, bx_v)
    pltpu.sync_copy(lvl, lvl_v)
    pltpu.sync_copy(gy, gy_v)
    pltpu.sync_copy(gx, gx_v)

    feats = (f2, f3, f4, f5)

    def box_body(i, carry):
        b = wid + i * _NW
        lvl_s = lvl_v[pl.ds(b, 16)][0]
        wi = jnp.int32(256) >> lvl_s
        wm1i = wi - 1
        wm1f = wm1i.astype(jnp.float32)
        bv = bx_v[pl.ds(b * 4, 16)]
        y1s = bv[0]
        x1s = bv[1]
        y2s = bv[2]
        x2s = bv[3]
        dys = y2s - y1s
        dxs = x2s - x1s

        for base in _GROUP_BASES:
            sl = pl.ds(base, 16)
            gyf = gy_v[sl]
            gxf = gx_v[sl]
            ys = y1s * wm1f + (gyf * dys) * wm1f
            xs = x1s * wm1f + (gxf * dxs) * wm1f
            y0 = ys.astype(jnp.int32)
            x0 = xs.astype(jnp.int32)
            wy = ys - y0.astype(jnp.float32)
            wx = xs - x0.astype(jnp.float32)
            y0c = jnp.minimum(y0, wm1i)
            x0c = jnp.minimum(x0, wm1i)
            y1c = jnp.minimum(y0c + 1, wm1i)
            x1c = jnp.minimum(x0c + 1, wm1i)
            cy = 1.0 - wy
            cx = 1.0 - wx
            yb0 = y0c * wi
            yb1 = y1c * wi
            idx_b[0, sl] = yb0 + x0c
            idx_b[1, sl] = yb0 + x1c
            idx_b[2, sl] = yb1 + x0c
            idx_b[3, sl] = yb1 + x1c
            wt_b[0, sl] = cy * cx
            wt_b[1, sl] = cy * wx
            wt_b[2, sl] = wy * cx
            wt_b[3, sl] = wy * wx

        for lev in range(4):
            @pl.when(lvl_s == lev)
            def _():
                cps = [pltpu.async_copy(feats[lev].at[idx_b.at[k]],
                                        crn.at[k], sem)
                       for k in range(4)]
                for cp in cps:
                    cp.wait()

        def s_body(s, c2):
            wtl = wt_b[0, pl.ds(s, 16)][0]
            wtr = wt_b[1, pl.ds(s, 16)][0]
            wbl = wt_b[2, pl.ds(s, 16)][0]
            wbr = wt_b[3, pl.ds(s, 16)][0]
            for c in range(_C // 16):
                cl = pl.ds(c * 16, 16)
                out_v[s, cl] = (crn[0, s, cl] * wtl + crn[1, s, cl] * wtr
                                + crn[2, s, cl] * wbl + crn[3, s, cl] * wbr)
            return c2

        lax.fori_loop(0, _S, s_body, 0)
        pltpu.sync_copy(out_v, out.at[b])
        return carry

    nb = (_N + _NW - 1 - wid) // _NW
    lax.fori_loop(0, nb, box_body, 0)


@functools.partial(jax.jit, static_argnums=())
def kernel(boxes, feat2, feat3, feat4, feat5):
    b = boxes[0]
    y1, x1, y2, x2 = b[:, 0], b[:, 1], b[:, 2], b[:, 3]
    h = y2 - y1
    w = x2 - x1
    image_area = 1024.0 * 1024.0
    roi_level = jnp.log(jnp.sqrt(h * w) / (224.0 / np.sqrt(image_area))) / np.log(2.0)
    lvl = jnp.clip(4 + jnp.round(roi_level).astype(jnp.int32), 2, 5) - 2

    lvl_pad = jnp.zeros((_NPAD,), jnp.int32).at[:_N].set(lvl)
    bx = jnp.zeros((4 * _NPAD,), jnp.float32).at[:4 * _N].set(b.reshape(-1))

    grid = jnp.arange(7, dtype=jnp.float32) / float(7 - 1)
    s_ids = np.minimum(np.arange(_SPAD), _S - 1)
    gy = grid[s_ids // 7]
    gx = grid[s_ids % 7]

    feats = [feat2[0].reshape(-1, _C), feat3[0].reshape(-1, _C),
             feat4[0].reshape(-1, _C), feat5[0].reshape(-1, _C)]

    mesh = plsc.VectorSubcoreMesh(core_axis_name="c", subcore_axis_name="s")
    out = pl.kernel(
        _body,
        out_type=jax.ShapeDtypeStruct((_N, _S, _C), jnp.float32),
        mesh=mesh,
        scratch_types=[
            pltpu.VMEM((4 * _NPAD,), jnp.float32),
            pltpu.VMEM((_NPAD,), jnp.int32),
            pltpu.VMEM((_SPAD,), jnp.float32),
            pltpu.VMEM((_SPAD,), jnp.float32),
            pltpu.VMEM((4, _SPAD), jnp.int32),
            pltpu.VMEM((4, 64), jnp.float32),
            pltpu.VMEM((4, _SPAD, _C), jnp.float32),
            pltpu.VMEM((_S, _C), jnp.float32),
            pltpu.SemaphoreType.DMA,
        ],
    )(bx, lvl_pad, gy, gx, *feats)
    return out.reshape(1, _N, 7, 7, _C)
